# baseline (speedup 1.0000x reference)
import functools

import jax
import jax.numpy as jnp
from jax import lax
from jax.experimental import pallas as pl
from jax.experimental.pallas import tpu as pltpu
from jax.experimental.pallas import tpu_sc as plsc

N_NODES = 10000
D = 128
B = 320000
NC, NS, L = 2, 16, 16
NW = NC * NS
EPW = B // NW
C = 80
NCH = EPW // C
G = C // L

_mesh = plsc.VectorSubcoreMesh(core_axis_name="c", subcore_axis_name="s")


@functools.partial(
    pl.kernel,
    out_type=jax.ShapeDtypeStruct((B,), jnp.float32),
    mesh=_mesh,
    scratch_types=[
        pltpu.VMEM_SHARED((N_NODES, D), jnp.float32),
        pltpu.VMEM((2 * C,), jnp.int32),
        pltpu.VMEM((2 * C,), jnp.int32),
        pltpu.VMEM((2 * C, D), jnp.float32),
        pltpu.VMEM((2 * C, D), jnp.float32),
        pltpu.VMEM((2 * C,), jnp.float32),
        [pltpu.SemaphoreType.DMA] * 2,
        [pltpu.SemaphoreType.DMA] * 2,
        [pltpu.SemaphoreType.DMA] * 2,
    ],
    compiler_params=pltpu.CompilerParams(
        needs_layout_passes=False,
        disable_bounds_checks=True,
    ),
)
def _link_classifier(table, src_idx, dst_idx, out_hbm,
                     table_sh, isx, idx, rsx, rdx, osx, semA, semB, semO):
    sid = lax.axis_index("s")
    wid = sid * NC + lax.axis_index("c")
    base = wid * EPW

    RPT = 640
    soff = jnp.minimum(sid * RPT, N_NODES - RPT)
    pltpu.sync_copy(table.at[pl.ds(soff, RPT)],
                    table_sh.at[pl.ds(soff, RPT)])

    def _slot_ops(i, s):
        co = base + i * C
        so = s * C
        a = (pltpu.make_async_copy(src_idx.at[pl.ds(co, C)],
                                   isx.at[pl.ds(so, C)], semA[s]),
             pltpu.make_async_copy(dst_idx.at[pl.ds(co, C)],
                                   idx.at[pl.ds(so, C)], semA[s]))
        b = (pltpu.make_async_copy(table.at[isx.at[pl.ds(so, C)]],
                                   rsx.at[pl.ds(so, C)], semB[s]),
             pltpu.make_async_copy(table.at[idx.at[pl.ds(so, C)]],
                                   rdx.at[pl.ds(so, C)], semB[s]))
        o = pltpu.make_async_copy(osx.at[pl.ds(so, C)],
                                  out_hbm.at[pl.ds(co, C)], semO[s])
        return a, b, o

    def _on_slot(slot, i, fn):
        if isinstance(slot, int):
            fn(*_slot_ops(i, slot))
            return

        @pl.when(slot == 0)
        def _():
            fn(*_slot_ops(i, 0))

        @pl.when(slot == 1)
        def _():
            fn(*_slot_ops(i, 1))

    def idx_copy(i, slot):
        _on_slot(slot, i, lambda a, b, o: (a[0].start(), a[1].start()))

    def idx_wait(i, slot):
        _on_slot(slot, i, lambda a, b, o: (a[0].wait(), a[1].wait()))

    def gather(i, slot):
        _on_slot(slot, i, lambda a, b, o: (b[0].start(), b[1].start()))

    def gather_wait(i, slot):
        _on_slot(slot, i, lambda a, b, o: (b[0].wait(), b[1].wait()))

    def out_write(i, slot):
        _on_slot(slot, i, lambda a, b, o: o.start())

    def out_wait(i, slot):
        _on_slot(slot, i, lambda a, b, o: o.wait())

    idx15 = jnp.full((L, 1), L - 1, jnp.int32)
    _gd = lax.GatherDimensionNumbers(
        offset_dims=(), collapsed_slice_dims=(0,), start_index_map=(0,))

    def _bcast_last(v):
        return lax.gather(v, idx15, _gd, slice_sizes=(1,),
                          mode=lax.GatherScatterMode.PROMISE_IN_BOUNDS)

    onehots = [
        (lax.iota(jnp.int32, L) == ee).astype(jnp.float32)
        for ee in range(L)
    ]

    def compute(slot):
        so = slot * C

        def e_body(eb, carry):
            eo = so + eb * L
            acc = None
            for ee in range(L):
                e = eo + ee
                prods = []
                for d in range(D // L):
                    a = rsx[e, pl.ds(d * L, L)]
                    b = rdx[e, pl.ds(d * L, L)]
                    prods.append(a * b)
                while len(prods) > 1:
                    prods = [x + y for x, y in zip(prods[::2], prods[1::2])]
                cum = jnp.cumsum(prods[0])
                t = _bcast_last(cum) * onehots[ee]
                acc = t if acc is None else acc + t
            osx[pl.ds(eo, L)] = acc
            return carry

        lax.fori_loop(0, G, e_body, 0)

    idx_copy(0, 0)
    idx_copy(1, 1)
    plsc.subcore_barrier()
    idx_wait(0, 0)
    gather(0, 0)

    def body(i, carry):
        slot = jnp.bitwise_and(i, 1)
        other = 1 - slot

        @pl.when(i + 1 < NCH)
        def _():
            idx_wait(i + 1, other)
            gather(i + 1, other)
        gather_wait(i, slot)

        @pl.when(i + 2 < NCH)
        def _():
            idx_copy(i + 2, slot)

        @pl.when(i >= 2)
        def _():
            out_wait(i - 2, slot)
        compute(slot)
        out_write(i, slot)
        return carry

    lax.fori_loop(0, NCH, body, 0)
    out_wait(NCH - 2, (NCH - 2) % 2)
    out_wait(NCH - 1, (NCH - 1) % 2)


@jax.jit
def kernel(embedding, edge_label_index):
    idx = edge_label_index.astype(jnp.int32)
    return _link_classifier(embedding, idx[0], idx[1])


# speedup vs baseline: 1.0453x; 1.0453x over previous
import functools

import jax
import jax.numpy as jnp
from jax import lax
from jax.experimental import pallas as pl
from jax.experimental.pallas import tpu as pltpu
from jax.experimental.pallas import tpu_sc as plsc

N_NODES = 10000
D = 128
B = 320000
NC, NS, L = 2, 16, 16
NW = NC * NS
EPW = B // NW
C = 80
NCH = EPW // C
G = C // L

_mesh = plsc.VectorSubcoreMesh(core_axis_name="c", subcore_axis_name="s")


@functools.partial(
    pl.kernel,
    out_type=jax.ShapeDtypeStruct((B,), jnp.float32),
    mesh=_mesh,
    scratch_types=[
        pltpu.VMEM((EPW,), jnp.int32),
        pltpu.VMEM((EPW,), jnp.int32),
        pltpu.VMEM((2 * C, D), jnp.float32),
        pltpu.VMEM((2 * C, D), jnp.float32),
        pltpu.VMEM((EPW,), jnp.float32),
        [pltpu.SemaphoreType.DMA] * 2,
    ],
    compiler_params=pltpu.CompilerParams(
        needs_layout_passes=False,
        disable_bounds_checks=True,
    ),
)
def _link_classifier(table, src_idx, dst_idx, out_hbm,
                     idx_s, idx_d, rsx, rdx, out_v, semB):
    sid = lax.axis_index("s")
    wid = sid * NC + lax.axis_index("c")
    base = wid * EPW

    pltpu.sync_copy(src_idx.at[pl.ds(base, EPW)], idx_s)
    pltpu.sync_copy(dst_idx.at[pl.ds(base, EPW)], idx_d)

    def _gathers(i, s):
        co = i * C
        so = s * C
        return (pltpu.make_async_copy(table.at[idx_s.at[pl.ds(co, C)]],
                                      rsx.at[pl.ds(so, C)], semB[s]),
                pltpu.make_async_copy(table.at[idx_d.at[pl.ds(co, C)]],
                                      rdx.at[pl.ds(so, C)], semB[s]))

    def _on_slot(slot, i, fn):
        if isinstance(slot, int):
            fn(*_gathers(i, slot))
            return

        @pl.when(slot == 0)
        def _():
            fn(*_gathers(i, 0))

        @pl.when(slot == 1)
        def _():
            fn(*_gathers(i, 1))

    def gather(i, slot):
        _on_slot(slot, i, lambda g0, g1: (g0.start(), g1.start()))

    def gather_wait(i, slot):
        _on_slot(slot, i, lambda g0, g1: (g0.wait(), g1.wait()))

    idx15 = jnp.full((L, 1), L - 1, jnp.int32)
    _gd = lax.GatherDimensionNumbers(
        offset_dims=(), collapsed_slice_dims=(0,), start_index_map=(0,))

    def _bcast_last(v):
        return lax.gather(v, idx15, _gd, slice_sizes=(1,),
                          mode=lax.GatherScatterMode.PROMISE_IN_BOUNDS)

    onehots = [
        (lax.iota(jnp.int32, L) == ee).astype(jnp.float32)
        for ee in range(L)
    ]

    def compute(i, slot):
        so = slot * C
        co = i * C

        def e_body(eb, carry):
            eo = so + eb * L
            acc = None
            for ee in range(L):
                e = eo + ee
                prods = []
                for d in range(D // L):
                    a = rsx[e, pl.ds(d * L, L)]
                    b = rdx[e, pl.ds(d * L, L)]
                    prods.append(a * b)
                while len(prods) > 1:
                    prods = [x + y for x, y in zip(prods[::2], prods[1::2])]
                cum = jnp.cumsum(prods[0])
                t = _bcast_last(cum) * onehots[ee]
                acc = t if acc is None else acc + t
            out_v[pl.ds(co + eb * L, L)] = acc
            return carry

        lax.fori_loop(0, G, e_body, 0)

    gather(0, 0)

    def body(i, carry):
        slot = jnp.bitwise_and(i, 1)

        @pl.when(i + 1 < NCH)
        def _():
            gather(i + 1, 1 - slot)
        gather_wait(i, slot)
        compute(i, slot)
        return carry

    lax.fori_loop(0, NCH, body, 0)
    pltpu.sync_copy(out_v, out_hbm.at[pl.ds(base, EPW)])


@jax.jit
def kernel(embedding, edge_label_index):
    idx = edge_label_index.astype(jnp.int32)
    return _link_classifier(embedding, idx[0], idx[1])
